# Initial kernel scaffold; baseline (speedup 1.0000x reference)
import jax
import jax.numpy as jnp
from jax.experimental import pallas as pl


def kernel(feat, intend, last_nodes, position_weight, segment_ids, gamma, beta, W_u, W_v, b_v, W_i, b_i, W_e):
    raise NotImplementedError("write your pallas kernel here")


# speedup vs baseline: 4.7869x; 4.7869x over previous
import functools

import jax
import jax.numpy as jnp
from jax import lax
from jax.experimental import pallas as pl
from jax.experimental.pallas import tpu as pltpu
from jax.experimental.pallas import tpu_sc as plsc


def _block_rows(n, want):
    best = None
    for k in range(8, want + 1, 8):
        if n % k == 0:
            best = k
    return best


def _make_sc_gather(n, d, b):
    info = plsc.get_sparse_core_info()
    nc, ns = info.num_cores, info.num_subcores
    nw = nc * ns
    assert b % (8 * nw) == 0 and d % info.num_lanes == 0
    b_per_w = b // nw
    mesh = plsc.VectorSubcoreMesh(core_axis_name="c", subcore_axis_name="s")

    @functools.partial(
        pl.kernel,
        mesh=mesh,
        out_type=jax.ShapeDtypeStruct((b, d), jnp.float32),
        scratch_types=[
            pltpu.VMEM((b_per_w,), jnp.int32),
            pltpu.VMEM((b_per_w, d), jnp.float32),
            pltpu.SemaphoreType.DMA,
        ],
    )
    def gather_k(table_hbm, idx_hbm, out_hbm, idx_v, rows_v, sem):
        wid = lax.axis_index("s") * nc + lax.axis_index("c")
        base = wid * b_per_w
        pltpu.sync_copy(idx_hbm.at[pl.ds(base, b_per_w)], idx_v)
        pltpu.async_copy(table_hbm.at[idx_v], rows_v, sem).wait()
        pltpu.sync_copy(rows_v, out_hbm.at[pl.ds(base, b_per_w)])

    return gather_k


def _stats_body(n, nb, feat_ref, g_ref, be_ref, ab_ref, acc_ref):
    i = pl.program_id(0)

    @pl.when(i == 0)
    def _init():
        acc_ref[...] = jnp.zeros_like(acc_ref)

    x = feat_ref[...]
    acc_ref[0:1, :] += jnp.sum(x, axis=0, keepdims=True)
    acc_ref[1:2, :] += jnp.sum(x * x, axis=0, keepdims=True)

    @pl.when(i == nb - 1)
    def _fin():
        mean = acc_ref[0:1, :] * (1.0 / n)
        var = acc_ref[1:2, :] * (1.0 / n) - mean * mean
        a = g_ref[...] * lax.rsqrt(var + 1e-5)
        ab_ref[0:1, :] = a
        ab_ref[1:2, :] = be_ref[...] - mean * a


def _q_body(fl_ref, it_ref, ab_ref, wv_ref, bv_ref, wi_ref, bi_ref, q_ref):
    fln = fl_ref[...] * ab_ref[0:1, :] + ab_ref[1:2, :]
    q_ref[...] = (
        jnp.dot(it_ref[...], wv_ref[...], preferred_element_type=jnp.float32)
        + bv_ref[...]
        + jnp.dot(fln, wi_ref[...], preferred_element_type=jnp.float32)
        + bi_ref[...]
    )


def _main_body(k, b, d, nb, seg_ref, pw_ref, feat_ref, ab_ref, q_ref, wu_ref,
               we_ref, rst_ref, pos_ref, acc_ref, accs_ref):
    i = pl.program_id(0)

    @pl.when(i == 0)
    def _init():
        acc_ref[...] = jnp.zeros_like(acc_ref)
        accs_ref[...] = jnp.zeros_like(accs_ref)

    x = feat_ref[...]
    xn = x * ab_ref[0:1, :] + ab_ref[1:2, :]
    u = jnp.dot(xn, wu_ref[...], preferred_element_type=jnp.float32)
    seg = seg_ref[0]
    oh = (lax.broadcasted_iota(jnp.int32, (b, k), 0)
          == jnp.broadcast_to(seg, (b, k))).astype(jnp.bfloat16)
    qn = lax.dot_general(oh, q_ref[...].astype(jnp.bfloat16),
                         dimension_numbers=(((0,), (0,)), ((), ())),
                         preferred_element_type=jnp.float32)
    e = jnp.dot(jax.nn.sigmoid(u + qn), we_ref[...],
                preferred_element_type=jnp.float32)
    w = jnp.exp(e)
    pw = pw_ref[0]
    y = jnp.concatenate([xn * w, xn * pw], axis=1).astype(jnp.bfloat16)
    acc_ref[...] += jnp.dot(oh, y, preferred_element_type=jnp.float32)
    accs_ref[...] += jnp.dot(oh, w.astype(jnp.bfloat16),
                             preferred_element_type=jnp.float32)

    @pl.when(i == nb - 1)
    def _fin():
        s = accs_ref[...]
        acc = acc_ref[...]
        rst_ref[...] = jnp.where(s > 0, acc[:, :d] / s, 0.0)
        pos_ref[...] = acc[:, d:]


def kernel(feat, intend, last_nodes, position_weight, segment_ids, gamma,
           beta, W_u, W_v, b_v, W_i, b_i, W_e):
    n, d = feat.shape
    b, h = intend.shape[0], W_u.shape[1]

    fl = _make_sc_gather(n, d, b)(feat, last_nodes.astype(jnp.int32))

    k1 = _block_rows(n, 2048)
    nb1 = n // k1
    ab = pl.pallas_call(
        functools.partial(_stats_body, n, nb1),
        grid=(nb1,),
        in_specs=[
            pl.BlockSpec((k1, d), lambda i: (i, 0)),
            pl.BlockSpec((1, d), lambda i: (0, 0)),
            pl.BlockSpec((1, d), lambda i: (0, 0)),
        ],
        out_specs=pl.BlockSpec((2, d), lambda i: (0, 0)),
        out_shape=jax.ShapeDtypeStruct((2, d), jnp.float32),
        scratch_shapes=[pltpu.VMEM((2, d), jnp.float32)],
    )(feat, gamma.reshape(1, d), beta.reshape(1, d))

    q = pl.pallas_call(
        _q_body,
        in_specs=[pl.BlockSpec((b, d), lambda: (0, 0)),
                  pl.BlockSpec((b, d), lambda: (0, 0)),
                  pl.BlockSpec((2, d), lambda: (0, 0)),
                  pl.BlockSpec((d, h), lambda: (0, 0)),
                  pl.BlockSpec((1, h), lambda: (0, 0)),
                  pl.BlockSpec((d, h), lambda: (0, 0)),
                  pl.BlockSpec((1, h), lambda: (0, 0))],
        out_specs=pl.BlockSpec((b, h), lambda: (0, 0)),
        out_shape=jax.ShapeDtypeStruct((b, h), jnp.float32),
    )(fl, intend, ab, W_v, b_v.reshape(1, h), W_i, b_i.reshape(1, h))

    k = _block_rows(n, 1024)
    nb = n // k
    seg3 = segment_ids.astype(jnp.int32).reshape(nb, 1, k)
    pw3 = position_weight.reshape(nb, k, 1)
    rst, pos = pl.pallas_call(
        functools.partial(_main_body, k, b, d, nb),
        grid=(nb,),
        in_specs=[
            pl.BlockSpec((1, 1, k), lambda i: (i, 0, 0)),
            pl.BlockSpec((1, k, 1), lambda i: (i, 0, 0)),
            pl.BlockSpec((k, d), lambda i: (i, 0)),
            pl.BlockSpec((2, d), lambda i: (0, 0)),
            pl.BlockSpec((b, h), lambda i: (0, 0)),
            pl.BlockSpec((d, h), lambda i: (0, 0)),
            pl.BlockSpec((h, 1), lambda i: (0, 0)),
        ],
        out_specs=[pl.BlockSpec((b, d), lambda i: (0, 0)),
                   pl.BlockSpec((b, d), lambda i: (0, 0))],
        out_shape=[jax.ShapeDtypeStruct((b, d), jnp.float32),
                   jax.ShapeDtypeStruct((b, d), jnp.float32)],
        scratch_shapes=[pltpu.VMEM((b, 2 * d), jnp.float32),
                        pltpu.VMEM((b, 1), jnp.float32)],
    )(seg3, pw3, feat, ab, q, W_u, W_e)

    return (rst, pos)


# speedup vs baseline: 9.5401x; 1.9930x over previous
import functools

import jax
import jax.numpy as jnp
from jax import lax
from jax.experimental import pallas as pl
from jax.experimental.pallas import tpu as pltpu
from jax.experimental.pallas import tpu_sc as plsc


def _block_rows(n, want):
    best = None
    for k in range(8, want + 1, 8):
        if n % k == 0:
            best = k
    return best


def _make_sc_gather(n, d, b):
    info = plsc.get_sparse_core_info()
    nc, ns = info.num_cores, info.num_subcores
    nw = nc * ns
    assert b % (8 * nw) == 0 and d % info.num_lanes == 0
    b_per_w = b // nw
    mesh = plsc.VectorSubcoreMesh(core_axis_name="c", subcore_axis_name="s")

    @functools.partial(
        pl.kernel,
        mesh=mesh,
        out_type=jax.ShapeDtypeStruct((b, d), jnp.float32),
        scratch_types=[
            pltpu.VMEM((b_per_w,), jnp.int32),
            pltpu.VMEM((b_per_w, d), jnp.float32),
            pltpu.SemaphoreType.DMA,
        ],
    )
    def gather_k(table_hbm, idx_hbm, out_hbm, idx_v, rows_v, sem):
        wid = lax.axis_index("s") * nc + lax.axis_index("c")
        base = wid * b_per_w
        pltpu.sync_copy(idx_hbm.at[pl.ds(base, b_per_w)], idx_v)
        pltpu.async_copy(table_hbm.at[idx_v], rows_v, sem).wait()
        pltpu.sync_copy(rows_v, out_hbm.at[pl.ds(base, b_per_w)])

    return gather_k


def _stats_body(n, nb, feat_ref, g_ref, be_ref, ab_ref, acc_ref):
    i = pl.program_id(0)

    @pl.when(i == 0)
    def _init():
        acc_ref[...] = jnp.zeros_like(acc_ref)

    x = feat_ref[...]
    acc_ref[0:1, :] += jnp.sum(x, axis=0, keepdims=True)
    acc_ref[1:2, :] += jnp.sum(x * x, axis=0, keepdims=True)

    @pl.when(i == nb - 1)
    def _fin():
        mean = acc_ref[0:1, :] * (1.0 / n)
        var = acc_ref[1:2, :] * (1.0 / n) - mean * mean
        a = g_ref[...] * lax.rsqrt(var + 1e-5)
        ab_ref[0:1, :] = a
        ab_ref[1:2, :] = be_ref[...] - mean * a


def _q_body(b, fl_ref, it_ref, ab_ref, wv_ref, bv_ref, wi_ref, bi_ref, q_ref):
    fln = fl_ref[...] * ab_ref[0:1, :] + ab_ref[1:2, :]
    q_ref[...] = jnp.zeros_like(q_ref)
    q_ref[0:b, :] = (
        jnp.dot(it_ref[...], wv_ref[...], preferred_element_type=jnp.float32)
        + bv_ref[...]
        + jnp.dot(fln, wi_ref[...], preferred_element_type=jnp.float32)
        + bi_ref[...]
    ).astype(jnp.bfloat16)


def _main_body(k, b, bp, d, w, nb, s0_ref, s1_ref, seg_ref, pw_ref, feat_ref,
               ab_ref, q_ref, wu_ref, we_ref, rst_ref, pos_ref, acc_ref,
               accs_ref):
    i = pl.program_id(0)

    @pl.when(i == 0)
    def _init():
        acc_ref[...] = jnp.zeros_like(acc_ref)
        accs_ref[...] = jnp.zeros_like(accs_ref)

    x = feat_ref[...]
    xn = x * ab_ref[0:1, :] + ab_ref[1:2, :]
    u = jnp.dot(xn, wu_ref[...], preferred_element_type=jnp.float32)
    seg = seg_ref[0]
    pw = pw_ref[0]
    s_lo = s0_ref[i]
    s8 = (s_lo // 8) * 8
    span_ok = (s1_ref[i] - s8) < (w - 1)

    def _attn(oh, qb):
        qn = lax.dot_general(oh, qb,
                             dimension_numbers=(((0,), (0,)), ((), ())),
                             preferred_element_type=jnp.float32)
        e = jnp.dot(jax.nn.sigmoid(u + qn), we_ref[...],
                    preferred_element_type=jnp.float32)
        wgt = jnp.exp(e)
        y = jnp.concatenate([xn * wgt, xn * pw], axis=1).astype(jnp.bfloat16)
        return (jnp.dot(oh, y, preferred_element_type=jnp.float32),
                jnp.dot(oh, wgt.astype(jnp.bfloat16),
                        preferred_element_type=jnp.float32))

    @pl.when(span_ok)
    def _narrow():
        oh = (lax.broadcasted_iota(jnp.int32, (w, k), 0)
              == jnp.broadcast_to(seg - s8, (w, k))).astype(jnp.bfloat16)
        dy, ds = _attn(oh, q_ref[pl.ds(s8, w), :])
        acc_ref[pl.ds(s8, w), :] = acc_ref[pl.ds(s8, w), :] + dy
        accs_ref[pl.ds(s8, w), :] = accs_ref[pl.ds(s8, w), :] + ds

    @pl.when(jnp.logical_not(span_ok))
    def _wide():
        oh = (lax.broadcasted_iota(jnp.int32, (b, k), 0)
              == jnp.broadcast_to(seg, (b, k))).astype(jnp.bfloat16)
        dy, ds = _attn(oh, q_ref[0:b, :])
        acc_ref[0:b, :] = acc_ref[0:b, :] + dy
        accs_ref[0:b, :] = accs_ref[0:b, :] + ds

    @pl.when(i == nb - 1)
    def _fin():
        s = accs_ref[0:b, :]
        acc = acc_ref[0:b, :]
        rst_ref[...] = jnp.where(s > 0, acc[:, :d] / s, 0.0)
        pos_ref[...] = acc[:, d:]


def kernel(feat, intend, last_nodes, position_weight, segment_ids, gamma,
           beta, W_u, W_v, b_v, W_i, b_i, W_e):
    n, d = feat.shape
    b, h = intend.shape[0], W_u.shape[1]

    fl = _make_sc_gather(n, d, b)(feat, last_nodes.astype(jnp.int32))

    k1 = _block_rows(n, 2048)
    nb1 = n // k1
    ab = pl.pallas_call(
        functools.partial(_stats_body, n, nb1),
        grid=(nb1,),
        in_specs=[
            pl.BlockSpec((k1, d), lambda i: (i, 0)),
            pl.BlockSpec((1, d), lambda i: (0, 0)),
            pl.BlockSpec((1, d), lambda i: (0, 0)),
        ],
        out_specs=pl.BlockSpec((2, d), lambda i: (0, 0)),
        out_shape=jax.ShapeDtypeStruct((2, d), jnp.float32),
        scratch_shapes=[pltpu.VMEM((2, d), jnp.float32)],
    )(feat, gamma.reshape(1, d), beta.reshape(1, d))

    win = 128
    bp = b + win
    q = pl.pallas_call(
        functools.partial(_q_body, b),
        in_specs=[pl.BlockSpec((b, d), lambda: (0, 0)),
                  pl.BlockSpec((b, d), lambda: (0, 0)),
                  pl.BlockSpec((2, d), lambda: (0, 0)),
                  pl.BlockSpec((d, h), lambda: (0, 0)),
                  pl.BlockSpec((1, h), lambda: (0, 0)),
                  pl.BlockSpec((d, h), lambda: (0, 0)),
                  pl.BlockSpec((1, h), lambda: (0, 0))],
        out_specs=pl.BlockSpec((bp, h), lambda: (0, 0)),
        out_shape=jax.ShapeDtypeStruct((bp, h), jnp.bfloat16),
    )(fl, intend, ab, W_v, b_v.reshape(1, h), W_i, b_i.reshape(1, h))

    k = _block_rows(n, 4096)
    nb = n // k
    seg32 = segment_ids.astype(jnp.int32)
    seg3 = seg32.reshape(nb, 1, k)
    pw3 = position_weight.reshape(nb, k, 1)
    s_start = seg32[::k]
    s_end = seg32[k - 1::k]
    grid_spec = pltpu.PrefetchScalarGridSpec(
        num_scalar_prefetch=2,
        grid=(nb,),
        in_specs=[
            pl.BlockSpec((1, 1, k), lambda i, *_: (i, 0, 0)),
            pl.BlockSpec((1, k, 1), lambda i, *_: (i, 0, 0)),
            pl.BlockSpec((k, d), lambda i, *_: (i, 0)),
            pl.BlockSpec((2, d), lambda i, *_: (0, 0)),
            pl.BlockSpec((bp, h), lambda i, *_: (0, 0)),
            pl.BlockSpec((d, h), lambda i, *_: (0, 0)),
            pl.BlockSpec((h, 1), lambda i, *_: (0, 0)),
        ],
        out_specs=[pl.BlockSpec((b, d), lambda i, *_: (0, 0)),
                   pl.BlockSpec((b, d), lambda i, *_: (0, 0))],
        scratch_shapes=[pltpu.VMEM((bp, 2 * d), jnp.float32),
                        pltpu.VMEM((bp, 1), jnp.float32)],
    )
    rst, pos = pl.pallas_call(
        functools.partial(_main_body, k, b, bp, d, win, nb),
        grid_spec=grid_spec,
        out_shape=[jax.ShapeDtypeStruct((b, d), jnp.float32),
                   jax.ShapeDtypeStruct((b, d), jnp.float32)],
    )(s_start, s_end, seg3, pw3, feat, ab, q, W_u, W_e)

    return (rst, pos)
